# baseline (speedup 1.0000x reference)
import functools

import jax
import jax.numpy as jnp
from jax import lax
from jax.experimental import pallas as pl
from jax.experimental.pallas import tpu as pltpu
from jax.experimental.pallas import tpu_sc as plsc

B = 4096
C_IN = 19
C_OUT = 64
D = 128
N_MASK = C_OUT - C_IN
IN_W = C_IN * D
OUT_W = C_OUT * D
MASK_W = N_MASK * D

NC = 2
NS = 16
NW = NC * NS
ROWS_PER_W = B // NW
CHUNK = 8
N_CHUNKS = ROWS_PER_W // CHUNK


def _sc_body(inp_hbm, maskblk_hbm, out_hbm, mask_v, in_v):
    wid = lax.axis_index("s") * NC + lax.axis_index("c")
    base = wid * ROWS_PER_W
    pltpu.sync_copy(maskblk_hbm, mask_v)
    for c in range(N_CHUNKS):
        r0 = base + c * CHUNK
        pltpu.sync_copy(inp_hbm.at[pl.ds(r0, CHUNK)], in_v)
        pltpu.sync_copy(in_v, out_hbm.at[pl.ds(r0, CHUNK), pl.ds(0, IN_W)])
        pltpu.sync_copy(mask_v, out_hbm.at[pl.ds(r0, CHUNK), pl.ds(IN_W, MASK_W)])


_sc_call = pl.kernel(
    _sc_body,
    mesh=plsc.VectorSubcoreMesh(core_axis_name="c", subcore_axis_name="s"),
    out_type=jax.ShapeDtypeStruct((B, OUT_W), jnp.float32),
    scratch_types=[
        pltpu.VMEM((CHUNK, MASK_W), jnp.float32),
        pltpu.VMEM((CHUNK, IN_W), jnp.float32),
    ],
)


@jax.jit
def kernel(inp, mask_token):
    inp2 = inp.reshape(B, IN_W)
    maskblk = jnp.tile(mask_token.reshape(1, D), (CHUNK, N_MASK))
    out = _sc_call(inp2, maskblk)
    return out.reshape(B, C_OUT, D)
